# Initial kernel scaffold; baseline (speedup 1.0000x reference)
import jax
import jax.numpy as jnp
from jax.experimental import pallas as pl


def kernel(x, We1, be1, We2, be2, Wfi, bfi, Wfo, bfo, E1, E2, E3, E4, Wd1, bd1, Wd2, bd2):
    raise NotImplementedError("write your pallas kernel here")


# speedup vs baseline: 1.0388x; 1.0388x over previous
import functools

import jax
import jax.numpy as jnp
from jax.experimental import pallas as pl
from jax.experimental.pallas import tpu as pltpu

B, C, T, H, F, NE = 128, 263, 512, 512, 4, 1024
TQ = T // 4


def _leaky(v):
    return jnp.where(v >= 0, v, 0.2 * v)


def _dot(a, b, precision=None):
    return jax.lax.dot_general(a, b, (((1,), (0,)), ((), ())),
                               preferred_element_type=jnp.float32,
                               precision=precision)


def _sr(a):
    return jnp.concatenate([jnp.zeros((a.shape[0], 1), a.dtype), a[:, :-1]], axis=1)


def _sl(a):
    return jnp.concatenate([a[:, 1:], jnp.zeros((a.shape[0], 1), a.dtype)], axis=1)


def _body(xp_ref, we1, be1, we2, be2, wfit, bfi, wfot, bfo, e_all, et_all,
          wd1, bd1, wd2, bd2, y_ref, loss_ref, perp_ref, counts_scr, loss_scr):
    b = pl.program_id(0)

    @pl.when(b == 0)
    def _init():
        counts_scr[...] = jnp.zeros_like(counts_scr)
        loss_scr[...] = jnp.zeros_like(loss_scr)

    x0 = xp_ref[0, 0]
    x1 = xp_ref[0, 1]
    x2 = xp_ref[0, 2]
    x3 = xp_ref[0, 3]

    h1e = _leaky(_dot(we1[0], _sr(x3)) + _dot(we1[1], x0)
                 + _dot(we1[2], x1) + _dot(we1[3], x2) + be1[...])
    h1o = _leaky(_dot(we1[0], x1) + _dot(we1[1], x2)
                 + _dot(we1[2], x3) + _dot(we1[3], _sl(x0)) + be1[...])

    h = _leaky(_dot(we2[0], _sr(h1o)) + _dot(we2[1], h1e)
               + _dot(we2[2], h1o) + _dot(we2[3], _sl(h1e)) + be2[...])

    ht = jnp.transpose(h)
    zp = _dot(ht, wfit[...]) + bfi[...]
    zh = jnp.round(zp)
    z_fsq = _dot(zh, wfot[...]) + bfo[...]

    r = ht - z_fsq
    loss_step = jnp.zeros((1, 1), jnp.float32)
    iota = jax.lax.broadcasted_iota(jnp.int32, (TQ, NE), 1)
    for k in range(4):
        e = e_all[k]
        en2 = jnp.sum(e * e, axis=1)
        rn2 = jnp.sum(r * r, axis=1, keepdims=True)
        d = (rn2 + en2[None, :]) - 2.0 * _dot(r, et_all[k])
        m = jnp.min(d, axis=1, keepdims=True)
        idxv = jnp.min(jnp.where(d == m, iota, NE), axis=1, keepdims=True)
        onehot = (iota == idxv).astype(jnp.float32)
        q = _dot(onehot, e)
        loss_step += jnp.sum(m, axis=0, keepdims=True)
        counts_scr[k] = counts_scr[k] + jnp.sum(onehot, axis=0, keepdims=True)
        r = r - q
    zq = h - jnp.transpose(r)
    loss_scr[...] = loss_scr[...] + loss_step

    y1e = _leaky(_dot(wd1[1], zq) + _dot(wd1[3], _sr(zq)) + bd1[...])
    y1o = _leaky(_dot(wd1[0], _sl(zq)) + _dot(wd1[2], zq) + bd1[...])

    y_ref[0, 0] = _dot(wd2[1], y1e) + _dot(wd2[3], _sr(y1o)) + bd2[...]
    y_ref[0, 1] = _dot(wd2[0], y1o) + _dot(wd2[2], y1e) + bd2[...]
    y_ref[0, 2] = _dot(wd2[1], y1o) + _dot(wd2[3], y1e) + bd2[...]
    y_ref[0, 3] = _dot(wd2[0], _sl(y1e)) + _dot(wd2[2], y1o) + bd2[...]

    @pl.when(b == B - 1)
    def _finish():
        loss_ref[...] = loss_scr[...] * (0.25 / (B * TQ * H))
        ptot = jnp.zeros((1, 1), jnp.float32)
        for k in range(4):
            avg = counts_scr[k] * (1.0 / (B * TQ))
            ent = jnp.sum(avg * jnp.log(avg + 1e-10), axis=1, keepdims=True)
            ptot += jnp.exp(-ent)
        perp_ref[...] = ptot * 0.25


@functools.partial(jax.jit, static_argnames=())
def kernel(x, We1, be1, We2, be2, Wfi, bfi, Wfo, bfo, E1, E2, E3, E4,
           Wd1, bd1, Wd2, bd2):
    xp = x.reshape(B, C, TQ, 4).transpose(0, 3, 1, 2)
    e_all = jnp.stack([E1, E2, E3, E4])
    et_all = e_all.transpose(0, 2, 1)
    we1 = We1.transpose(2, 0, 1)
    we2 = We2.transpose(2, 0, 1)
    wd1 = Wd1.transpose(2, 1, 0)
    wd2 = Wd2.transpose(2, 1, 0)

    def c2(v):
        return v.reshape(-1, 1)

    full = lambda s: pl.BlockSpec(s, lambda b: (0,) * len(s))
    out_y, out_loss, out_perp = pl.pallas_call(
        _body,
        grid=(B,),
        in_specs=[
            pl.BlockSpec((1, 4, C, TQ), lambda b: (b, 0, 0, 0)),
            full((4, H, C)), full((H, 1)),
            full((4, H, H)), full((H, 1)),
            full((H, F)), full((1, F)),
            full((F, H)), full((1, H)),
            full((4, NE, H)), full((4, H, NE)),
            full((4, H, H)), full((H, 1)),
            full((4, C, H)), full((C, 1)),
        ],
        out_specs=[
            pl.BlockSpec((1, 4, C, TQ), lambda b: (b, 0, 0, 0)),
            pl.BlockSpec((1, 1), lambda b: (0, 0)),
            pl.BlockSpec((1, 1), lambda b: (0, 0)),
        ],
        out_shape=[
            jax.ShapeDtypeStruct((B, 4, C, TQ), jnp.float32),
            jax.ShapeDtypeStruct((1, 1), jnp.float32),
            jax.ShapeDtypeStruct((1, 1), jnp.float32),
        ],
        scratch_shapes=[
            pltpu.VMEM((4, 1, NE), jnp.float32),
            pltpu.VMEM((1, 1), jnp.float32),
        ],
    )(xp, we1, c2(be1), we2, c2(be2), Wfi.T, bfi.reshape(1, F),
      Wfo.T, bfo.reshape(1, H), e_all, et_all, wd1, c2(bd1), wd2, c2(bd2))

    y = out_y.transpose(0, 2, 3, 1).reshape(B, C, T)
    return (y, out_loss[0, 0], out_perp[0, 0])


# speedup vs baseline: 1.0749x; 1.0348x over previous
import functools

import jax
import jax.numpy as jnp
from jax.experimental import pallas as pl
from jax.experimental.pallas import tpu as pltpu

B, C, T, H, F, NE = 128, 263, 512, 512, 4, 1024
TQ = T // 4
BB = 2


def _leaky(v):
    return jnp.where(v >= 0, v, 0.2 * v)


def _dot(a, b, precision=None):
    return jax.lax.dot_general(a, b, (((1,), (0,)), ((), ())),
                               preferred_element_type=jnp.float32,
                               precision=precision)


def _sr(a):
    return jnp.concatenate([jnp.zeros((a.shape[0], 1), a.dtype), a[:, :-1]], axis=1)


def _sl(a):
    return jnp.concatenate([a[:, 1:], jnp.zeros((a.shape[0], 1), a.dtype)], axis=1)


def _body(xp_ref, we1, be1, we2, be2, wfit, bfi, wfot, bfo, e_all, et_all,
          wd1, bd1, wd2, bd2, y_ref, loss_ref, perp_ref, counts_scr, loss_scr):
    b = pl.program_id(0)

    @pl.when(b == 0)
    def _init():
        counts_scr[...] = jnp.zeros_like(counts_scr)
        loss_scr[...] = jnp.zeros_like(loss_scr)

    loss_step = jnp.zeros((1, 1), jnp.float32)
    cnt = [counts_scr[k] for k in range(4)]
    iota = jax.lax.broadcasted_iota(jnp.int32, (TQ, NE), 1)
    en2s = [jnp.sum(e_all[k] * e_all[k], axis=1) for k in range(4)]

    for i in range(BB):
        x0 = xp_ref[i, 0]
        x1 = xp_ref[i, 1]
        x2 = xp_ref[i, 2]
        x3 = xp_ref[i, 3]

        h1e = _leaky(_dot(we1[0], _sr(x3)) + _dot(we1[1], x0)
                     + _dot(we1[2], x1) + _dot(we1[3], x2) + be1[...])
        h1o = _leaky(_dot(we1[0], x1) + _dot(we1[1], x2)
                     + _dot(we1[2], x3) + _dot(we1[3], _sl(x0)) + be1[...])

        h = _leaky(_dot(we2[0], _sr(h1o)) + _dot(we2[1], h1e)
                   + _dot(we2[2], h1o) + _dot(we2[3], _sl(h1e)) + be2[...])

        ht = jnp.transpose(h)
        zp = _dot(ht, wfit[...]) + bfi[...]
        zh = jnp.round(zp)
        z_fsq = _dot(zh, wfot[...]) + bfo[...]

        r = ht - z_fsq
        for k in range(4):
            rn2 = jnp.sum(r * r, axis=1, keepdims=True)
            d = (rn2 + en2s[k][None, :]) - 2.0 * _dot(r, et_all[k])
            m = jnp.min(d, axis=1, keepdims=True)
            idxv = jnp.min(jnp.where(d == m, iota, NE), axis=1, keepdims=True)
            onehot = (iota == idxv).astype(jnp.float32)
            q = _dot(onehot, e_all[k])
            loss_step += jnp.sum(m, axis=0, keepdims=True)
            cnt[k] = cnt[k] + jnp.sum(onehot, axis=0, keepdims=True)
            r = r - q
        zq = h - jnp.transpose(r)

        y1e = _leaky(_dot(wd1[1], zq) + _dot(wd1[3], _sr(zq)) + bd1[...])
        y1o = _leaky(_dot(wd1[0], _sl(zq)) + _dot(wd1[2], zq) + bd1[...])

        y_ref[i, 0] = _dot(wd2[1], y1e) + _dot(wd2[3], _sr(y1o)) + bd2[...]
        y_ref[i, 1] = _dot(wd2[0], y1o) + _dot(wd2[2], y1e) + bd2[...]
        y_ref[i, 2] = _dot(wd2[1], y1o) + _dot(wd2[3], y1e) + bd2[...]
        y_ref[i, 3] = _dot(wd2[0], _sl(y1e)) + _dot(wd2[2], y1o) + bd2[...]

    for k in range(4):
        counts_scr[k] = cnt[k]
    loss_scr[...] = loss_scr[...] + loss_step

    @pl.when(b == B // BB - 1)
    def _finish():
        loss_ref[...] = loss_scr[...] * (0.25 / (B * TQ * H))
        ptot = jnp.zeros((1, 1), jnp.float32)
        for k in range(4):
            avg = counts_scr[k] * (1.0 / (B * TQ))
            ent = jnp.sum(avg * jnp.log(avg + 1e-10), axis=1, keepdims=True)
            ptot += jnp.exp(-ent)
        perp_ref[...] = ptot * 0.25


@functools.partial(jax.jit, static_argnames=())
def kernel(x, We1, be1, We2, be2, Wfi, bfi, Wfo, bfo, E1, E2, E3, E4,
           Wd1, bd1, Wd2, bd2):
    xp = x.reshape(B, C, TQ, 4).transpose(0, 3, 1, 2)
    e_all = jnp.stack([E1, E2, E3, E4])
    et_all = e_all.transpose(0, 2, 1)
    we1 = We1.transpose(2, 0, 1)
    we2 = We2.transpose(2, 0, 1)
    wfit = Wfi.T
    wfot = Wfo.T
    bfip = bfi.reshape(1, F)
    wd1 = Wd1.transpose(2, 1, 0)
    wd2 = Wd2.transpose(2, 1, 0)

    def c2(v):
        return v.reshape(-1, 1)

    full = lambda s: pl.BlockSpec(s, lambda b: (0,) * len(s))
    out_y, out_loss, out_perp = pl.pallas_call(
        _body,
        grid=(B // BB,),
        in_specs=[
            pl.BlockSpec((BB, 4, C, TQ), lambda b: (b, 0, 0, 0)),
            full((4, H, C)), full((H, 1)),
            full((4, H, H)), full((H, 1)),
            full((H, F)), full((1, F)),
            full((F, H)), full((1, H)),
            full((4, NE, H)), full((4, H, NE)),
            full((4, H, H)), full((H, 1)),
            full((4, C, H)), full((C, 1)),
        ],
        out_specs=[
            pl.BlockSpec((BB, 4, C, TQ), lambda b: (b, 0, 0, 0)),
            pl.BlockSpec((1, 1), lambda b: (0, 0)),
            pl.BlockSpec((1, 1), lambda b: (0, 0)),
        ],
        out_shape=[
            jax.ShapeDtypeStruct((B, 4, C, TQ), jnp.float32),
            jax.ShapeDtypeStruct((1, 1), jnp.float32),
            jax.ShapeDtypeStruct((1, 1), jnp.float32),
        ],
        scratch_shapes=[
            pltpu.VMEM((4, 1, NE), jnp.float32),
            pltpu.VMEM((1, 1), jnp.float32),
        ],
    )(xp, we1, c2(be1), we2, c2(be2), wfit, bfip,
      wfot, bfo.reshape(1, H), e_all, et_all, wd1, c2(bd1), wd2, c2(bd2))

    y = out_y.transpose(0, 2, 3, 1).reshape(B, C, T)
    return (y, out_loss[0, 0], out_perp[0, 0])


# speedup vs baseline: 1.0863x; 1.0105x over previous
import functools

import jax
import jax.numpy as jnp
from jax.experimental import pallas as pl
from jax.experimental.pallas import tpu as pltpu

B, C, T, H, F, NE = 128, 263, 512, 512, 4, 1024
TQ = T // 4
BB = 4


def _leaky(v):
    return jnp.where(v >= 0, v, 0.2 * v)


def _dot(a, b, precision=None):
    return jax.lax.dot_general(a, b, (((1,), (0,)), ((), ())),
                               preferred_element_type=jnp.float32,
                               precision=precision)


def _sr(a):
    return jnp.concatenate([jnp.zeros((a.shape[0], 1), a.dtype), a[:, :-1]], axis=1)


def _sl(a):
    return jnp.concatenate([a[:, 1:], jnp.zeros((a.shape[0], 1), a.dtype)], axis=1)


def _body(xp_ref, we1, be1, we2, be2, wfit, bfi, wfot, bfo, e_all, et_all,
          wd1, bd1, wd2, bd2, y_ref, loss_ref, perp_ref, counts_scr, loss_scr):
    b = pl.program_id(0)

    @pl.when(b == 0)
    def _init():
        counts_scr[...] = jnp.zeros_like(counts_scr)
        loss_scr[...] = jnp.zeros_like(loss_scr)

    loss_step = jnp.zeros((1, 1), jnp.float32)
    cnt = [counts_scr[k] for k in range(4)]
    iota = jax.lax.broadcasted_iota(jnp.int32, (TQ, NE), 1)
    en2s = [jnp.sum(e_all[k] * e_all[k], axis=1) for k in range(4)]

    for i in range(BB):
        x0 = xp_ref[i, 0]
        x1 = xp_ref[i, 1]
        x2 = xp_ref[i, 2]
        x3 = xp_ref[i, 3]

        h1e = _leaky(_dot(we1[0], _sr(x3)) + _dot(we1[1], x0)
                     + _dot(we1[2], x1) + _dot(we1[3], x2) + be1[...])
        h1o = _leaky(_dot(we1[0], x1) + _dot(we1[1], x2)
                     + _dot(we1[2], x3) + _dot(we1[3], _sl(x0)) + be1[...])

        h = _leaky(_dot(we2[0], _sr(h1o)) + _dot(we2[1], h1e)
                   + _dot(we2[2], h1o) + _dot(we2[3], _sl(h1e)) + be2[...])

        ht = jnp.transpose(h)
        zp = _dot(ht, wfit[...]) + bfi[...]
        zh = jnp.round(zp)
        z_fsq = _dot(zh, wfot[...]) + bfo[...]

        r = ht - z_fsq
        for k in range(4):
            rn2 = jnp.sum(r * r, axis=1, keepdims=True)
            d = (rn2 + en2s[k][None, :]) - 2.0 * _dot(r, et_all[k])
            m = jnp.min(d, axis=1, keepdims=True)
            idxv = jnp.min(jnp.where(d == m, iota, NE), axis=1, keepdims=True)
            onehot = (iota == idxv).astype(jnp.float32)
            q = _dot(onehot, e_all[k])
            loss_step += jnp.sum(m, axis=0, keepdims=True)
            cnt[k] = cnt[k] + jnp.sum(onehot, axis=0, keepdims=True)
            r = r - q
        zq = h - jnp.transpose(r)

        y1e = _leaky(_dot(wd1[1], zq) + _dot(wd1[3], _sr(zq)) + bd1[...])
        y1o = _leaky(_dot(wd1[0], _sl(zq)) + _dot(wd1[2], zq) + bd1[...])

        y_ref[i, 0] = _dot(wd2[1], y1e) + _dot(wd2[3], _sr(y1o)) + bd2[...]
        y_ref[i, 1] = _dot(wd2[0], y1o) + _dot(wd2[2], y1e) + bd2[...]
        y_ref[i, 2] = _dot(wd2[1], y1o) + _dot(wd2[3], y1e) + bd2[...]
        y_ref[i, 3] = _dot(wd2[0], _sl(y1e)) + _dot(wd2[2], y1o) + bd2[...]

    for k in range(4):
        counts_scr[k] = cnt[k]
    loss_scr[...] = loss_scr[...] + loss_step

    @pl.when(b == B // BB - 1)
    def _finish():
        loss_ref[...] = loss_scr[...] * (0.25 / (B * TQ * H))
        ptot = jnp.zeros((1, 1), jnp.float32)
        for k in range(4):
            avg = counts_scr[k] * (1.0 / (B * TQ))
            ent = jnp.sum(avg * jnp.log(avg + 1e-10), axis=1, keepdims=True)
            ptot += jnp.exp(-ent)
        perp_ref[...] = ptot * 0.25


@functools.partial(jax.jit, static_argnames=())
def kernel(x, We1, be1, We2, be2, Wfi, bfi, Wfo, bfo, E1, E2, E3, E4,
           Wd1, bd1, Wd2, bd2):
    xp = x.reshape(B, C, TQ, 4).transpose(0, 3, 1, 2)
    e_all = jnp.stack([E1, E2, E3, E4])
    et_all = e_all.transpose(0, 2, 1)
    we1 = We1.transpose(2, 0, 1)
    we2 = We2.transpose(2, 0, 1)
    wfit = Wfi.T
    wfot = Wfo.T
    bfip = bfi.reshape(1, F)
    wd1 = Wd1.transpose(2, 1, 0)
    wd2 = Wd2.transpose(2, 1, 0)

    def c2(v):
        return v.reshape(-1, 1)

    full = lambda s: pl.BlockSpec(s, lambda b: (0,) * len(s))
    out_y, out_loss, out_perp = pl.pallas_call(
        _body,
        grid=(B // BB,),
        in_specs=[
            pl.BlockSpec((BB, 4, C, TQ), lambda b: (b, 0, 0, 0)),
            full((4, H, C)), full((H, 1)),
            full((4, H, H)), full((H, 1)),
            full((H, F)), full((1, F)),
            full((F, H)), full((1, H)),
            full((4, NE, H)), full((4, H, NE)),
            full((4, H, H)), full((H, 1)),
            full((4, C, H)), full((C, 1)),
        ],
        out_specs=[
            pl.BlockSpec((BB, 4, C, TQ), lambda b: (b, 0, 0, 0)),
            pl.BlockSpec((1, 1), lambda b: (0, 0)),
            pl.BlockSpec((1, 1), lambda b: (0, 0)),
        ],
        out_shape=[
            jax.ShapeDtypeStruct((B, 4, C, TQ), jnp.float32),
            jax.ShapeDtypeStruct((1, 1), jnp.float32),
            jax.ShapeDtypeStruct((1, 1), jnp.float32),
        ],
        scratch_shapes=[
            pltpu.VMEM((4, 1, NE), jnp.float32),
            pltpu.VMEM((1, 1), jnp.float32),
        ],
    )(xp, we1, c2(be1), we2, c2(be2), wfit, bfip,
      wfot, bfo.reshape(1, H), e_all, et_all, wd1, c2(bd1), wd2, c2(bd2))

    y = out_y.transpose(0, 2, 3, 1).reshape(B, C, T)
    return (y, out_loss[0, 0], out_perp[0, 0])


# speedup vs baseline: 1.6618x; 1.5298x over previous
import functools

import jax
import jax.numpy as jnp
from jax.experimental import pallas as pl
from jax.experimental.pallas import tpu as pltpu

B, C, T, H, F, NE = 128, 263, 512, 512, 4, 1024
TQ = T // 4
BB = 4


def _leaky(v):
    return jnp.where(v >= 0, v, 0.2 * v)


def _dot(a, b, precision=None):
    return jax.lax.dot_general(a, b, (((1,), (0,)), ((), ())),
                               preferred_element_type=jnp.float32,
                               precision=precision)


def _sr(a):
    return jnp.concatenate([jnp.zeros((a.shape[0], 1), a.dtype), a[:, :-1]], axis=1)


def _sl(a):
    return jnp.concatenate([a[:, 1:], jnp.zeros((a.shape[0], 1), a.dtype)], axis=1)


def _body(xp_ref, we1, be1, we2, be2, wfit, bfi, wfot, bfo, e_all, et_all,
          wd1, bd1, wd2, bd2, y_ref, loss_ref, perp_ref, counts_scr, loss_scr):
    b = pl.program_id(0)

    @pl.when(b == 0)
    def _init():
        counts_scr[...] = jnp.zeros_like(counts_scr)
        loss_scr[...] = jnp.zeros_like(loss_scr)

    loss_step = jnp.zeros((1, 1), jnp.float32)
    iota = jax.lax.broadcasted_iota(jnp.int32, (BB * TQ, NE), 1)
    en2s = [jnp.sum(e_all[k] * e_all[k], axis=1) for k in range(4)]

    def cat(f):
        return jnp.concatenate([f(i) for i in range(BB)], axis=1)

    def seg_sr(a):
        return jnp.concatenate(
            [_sr(a[:, i * TQ:(i + 1) * TQ]) for i in range(BB)], axis=1)

    def seg_sl(a):
        return jnp.concatenate(
            [_sl(a[:, i * TQ:(i + 1) * TQ]) for i in range(BB)], axis=1)

    x0 = cat(lambda i: xp_ref[i, 0])
    x1 = cat(lambda i: xp_ref[i, 1])
    x2 = cat(lambda i: xp_ref[i, 2])
    x3 = cat(lambda i: xp_ref[i, 3])
    h1e = _leaky(_dot(we1[0], seg_sr(x3)) + _dot(we1[1], x0)
                 + _dot(we1[2], x1) + _dot(we1[3], x2) + be1[...])
    h1o = _leaky(_dot(we1[0], x1) + _dot(we1[1], x2)
                 + _dot(we1[2], x3) + _dot(we1[3], seg_sl(x0)) + be1[...])

    h = _leaky(_dot(we2[0], seg_sr(h1o)) + _dot(we2[1], h1e)
               + _dot(we2[2], h1o) + _dot(we2[3], seg_sl(h1e)) + be2[...])

    ht = jnp.transpose(h)
    zp = _dot(ht, wfit[...]) + bfi[...]
    zh = jnp.round(zp)
    z_fsq = _dot(zh, wfot[...]) + bfo[...]

    r = ht - z_fsq
    for k in range(4):
        rn2 = jnp.sum(r * r, axis=1, keepdims=True)
        d = (rn2 + en2s[k][None, :]) - 2.0 * _dot(r, et_all[k])
        m = jnp.min(d, axis=1, keepdims=True)
        idxv = jnp.min(jnp.where(d == m, iota, NE), axis=1, keepdims=True)
        onehot = (iota == idxv).astype(jnp.float32)
        q = _dot(onehot, e_all[k])
        loss_step += jnp.sum(m, axis=0, keepdims=True)
        counts_scr[k] = counts_scr[k] + jnp.sum(onehot, axis=0, keepdims=True)
        r = r - q
    zq = h - jnp.transpose(r)

    y1e = _leaky(_dot(wd1[1], zq) + _dot(wd1[3], seg_sr(zq)) + bd1[...])
    y1o = _leaky(_dot(wd1[0], seg_sl(zq)) + _dot(wd1[2], zq) + bd1[...])

    y0 = _dot(wd2[1], y1e) + _dot(wd2[3], seg_sr(y1o)) + bd2[...]
    y1 = _dot(wd2[0], y1o) + _dot(wd2[2], y1e) + bd2[...]
    y2 = _dot(wd2[1], y1o) + _dot(wd2[3], y1e) + bd2[...]
    y3 = _dot(wd2[0], seg_sl(y1e)) + _dot(wd2[2], y1o) + bd2[...]
    for i in range(BB):
        y_ref[i, 0] = y0[:, i * TQ:(i + 1) * TQ]
        y_ref[i, 1] = y1[:, i * TQ:(i + 1) * TQ]
        y_ref[i, 2] = y2[:, i * TQ:(i + 1) * TQ]
        y_ref[i, 3] = y3[:, i * TQ:(i + 1) * TQ]

    loss_scr[...] = loss_scr[...] + loss_step

    @pl.when(b == B // BB - 1)
    def _finish():
        loss_ref[...] = loss_scr[...] * (0.25 / (B * TQ * H))
        ptot = jnp.zeros((1, 1), jnp.float32)
        for k in range(4):
            avg = counts_scr[k] * (1.0 / (B * TQ))
            ent = jnp.sum(avg * jnp.log(avg + 1e-10), axis=1, keepdims=True)
            ptot += jnp.exp(-ent)
        perp_ref[...] = ptot * 0.25


@functools.partial(jax.jit, static_argnames=())
def kernel(x, We1, be1, We2, be2, Wfi, bfi, Wfo, bfo, E1, E2, E3, E4,
           Wd1, bd1, Wd2, bd2):
    xp = x.reshape(B, C, TQ, 4).transpose(0, 3, 1, 2)
    e_all = jnp.stack([E1, E2, E3, E4])
    et_all = e_all.transpose(0, 2, 1)
    we1 = We1.transpose(2, 0, 1)
    we2 = We2.transpose(2, 0, 1)
    wfit = Wfi.T
    wfot = Wfo.T
    bfip = bfi.reshape(1, F)
    wd1 = Wd1.transpose(2, 1, 0)
    wd2 = Wd2.transpose(2, 1, 0)

    def c2(v):
        return v.reshape(-1, 1)

    full = lambda s: pl.BlockSpec(s, lambda b: (0,) * len(s))
    out_y, out_loss, out_perp = pl.pallas_call(
        _body,
        grid=(B // BB,),
        in_specs=[
            pl.BlockSpec((BB, 4, C, TQ), lambda b: (b, 0, 0, 0)),
            full((4, H, C)), full((H, 1)),
            full((4, H, H)), full((H, 1)),
            full((H, F)), full((1, F)),
            full((F, H)), full((1, H)),
            full((4, NE, H)), full((4, H, NE)),
            full((4, H, H)), full((H, 1)),
            full((4, C, H)), full((C, 1)),
        ],
        out_specs=[
            pl.BlockSpec((BB, 4, C, TQ), lambda b: (b, 0, 0, 0)),
            pl.BlockSpec((1, 1), lambda b: (0, 0)),
            pl.BlockSpec((1, 1), lambda b: (0, 0)),
        ],
        out_shape=[
            jax.ShapeDtypeStruct((B, 4, C, TQ), jnp.float32),
            jax.ShapeDtypeStruct((1, 1), jnp.float32),
            jax.ShapeDtypeStruct((1, 1), jnp.float32),
        ],
        scratch_shapes=[
            pltpu.VMEM((4, 1, NE), jnp.float32),
            pltpu.VMEM((1, 1), jnp.float32),
        ],
    )(xp, we1, c2(be1), we2, c2(be2), wfit, bfip,
      wfot, bfo.reshape(1, H), e_all, et_all, wd1, c2(bd1), wd2, c2(bd2))

    y = out_y.transpose(0, 2, 3, 1).reshape(B, C, T)
    return (y, out_loss[0, 0], out_perp[0, 0])


# speedup vs baseline: 1.7777x; 1.0698x over previous
import functools

import jax
import jax.numpy as jnp
from jax.experimental import pallas as pl
from jax.experimental.pallas import tpu as pltpu

B, C, T, H, F, NE = 128, 263, 512, 512, 4, 1024
TQ = T // 4
BB = 8


def _leaky(v):
    return jnp.where(v >= 0, v, 0.2 * v)


def _dot(a, b, precision=None):
    return jax.lax.dot_general(a, b, (((1,), (0,)), ((), ())),
                               preferred_element_type=jnp.float32,
                               precision=precision)


def _sr(a):
    return jnp.concatenate([jnp.zeros((a.shape[0], 1), a.dtype), a[:, :-1]], axis=1)


def _sl(a):
    return jnp.concatenate([a[:, 1:], jnp.zeros((a.shape[0], 1), a.dtype)], axis=1)


def _body(xp_ref, we1, be1, we2, be2, wfit, bfi, wfot, bfo, e_all, et_all,
          wd1, bd1, wd2, bd2, y_ref, loss_ref, perp_ref, counts_scr, loss_scr):
    b = pl.program_id(0)

    @pl.when(b == 0)
    def _init():
        counts_scr[...] = jnp.zeros_like(counts_scr)
        loss_scr[...] = jnp.zeros_like(loss_scr)

    loss_step = jnp.zeros((1, 1), jnp.float32)
    iota = jax.lax.broadcasted_iota(jnp.int32, (BB * TQ, NE), 1)
    en2s = [jnp.sum(e_all[k] * e_all[k], axis=1) for k in range(4)]

    def cat(f):
        return jnp.concatenate([f(i) for i in range(BB)], axis=1)

    def seg_sr(a):
        return jnp.concatenate(
            [_sr(a[:, i * TQ:(i + 1) * TQ]) for i in range(BB)], axis=1)

    def seg_sl(a):
        return jnp.concatenate(
            [_sl(a[:, i * TQ:(i + 1) * TQ]) for i in range(BB)], axis=1)

    x0 = cat(lambda i: xp_ref[i, 0])
    x1 = cat(lambda i: xp_ref[i, 1])
    x2 = cat(lambda i: xp_ref[i, 2])
    x3 = cat(lambda i: xp_ref[i, 3])
    h1e = _leaky(_dot(we1[0], seg_sr(x3)) + _dot(we1[1], x0)
                 + _dot(we1[2], x1) + _dot(we1[3], x2) + be1[...])
    h1o = _leaky(_dot(we1[0], x1) + _dot(we1[1], x2)
                 + _dot(we1[2], x3) + _dot(we1[3], seg_sl(x0)) + be1[...])

    h = _leaky(_dot(we2[0], seg_sr(h1o)) + _dot(we2[1], h1e)
               + _dot(we2[2], h1o) + _dot(we2[3], seg_sl(h1e)) + be2[...])

    ht = jnp.transpose(h)
    zp = _dot(ht, wfit[...]) + bfi[...]
    zh = jnp.round(zp)
    z_fsq = _dot(zh, wfot[...]) + bfo[...]

    r = ht - z_fsq
    for k in range(4):
        rn2 = jnp.sum(r * r, axis=1, keepdims=True)
        d = (rn2 + en2s[k][None, :]) - 2.0 * _dot(r, et_all[k])
        m = jnp.min(d, axis=1, keepdims=True)
        idxv = jnp.min(jnp.where(d == m, iota, NE), axis=1, keepdims=True)
        onehot = (iota == idxv).astype(jnp.float32)
        q = _dot(onehot, e_all[k])
        loss_step += jnp.sum(m, axis=0, keepdims=True)
        counts_scr[k] = counts_scr[k] + jnp.sum(onehot, axis=0, keepdims=True)
        r = r - q
    zq = h - jnp.transpose(r)

    y1e = _leaky(_dot(wd1[1], zq) + _dot(wd1[3], seg_sr(zq)) + bd1[...])
    y1o = _leaky(_dot(wd1[0], seg_sl(zq)) + _dot(wd1[2], zq) + bd1[...])

    y0 = _dot(wd2[1], y1e) + _dot(wd2[3], seg_sr(y1o)) + bd2[...]
    y1 = _dot(wd2[0], y1o) + _dot(wd2[2], y1e) + bd2[...]
    y2 = _dot(wd2[1], y1o) + _dot(wd2[3], y1e) + bd2[...]
    y3 = _dot(wd2[0], seg_sl(y1e)) + _dot(wd2[2], y1o) + bd2[...]
    for i in range(BB):
        y_ref[i, 0] = y0[:, i * TQ:(i + 1) * TQ]
        y_ref[i, 1] = y1[:, i * TQ:(i + 1) * TQ]
        y_ref[i, 2] = y2[:, i * TQ:(i + 1) * TQ]
        y_ref[i, 3] = y3[:, i * TQ:(i + 1) * TQ]

    loss_scr[...] = loss_scr[...] + loss_step

    @pl.when(b == B // BB - 1)
    def _finish():
        loss_ref[...] = loss_scr[...] * (0.25 / (B * TQ * H))
        ptot = jnp.zeros((1, 1), jnp.float32)
        for k in range(4):
            avg = counts_scr[k] * (1.0 / (B * TQ))
            ent = jnp.sum(avg * jnp.log(avg + 1e-10), axis=1, keepdims=True)
            ptot += jnp.exp(-ent)
        perp_ref[...] = ptot * 0.25


@functools.partial(jax.jit, static_argnames=())
def kernel(x, We1, be1, We2, be2, Wfi, bfi, Wfo, bfo, E1, E2, E3, E4,
           Wd1, bd1, Wd2, bd2):
    xp = x.reshape(B, C, TQ, 4).transpose(0, 3, 1, 2)
    e_all = jnp.stack([E1, E2, E3, E4])
    et_all = e_all.transpose(0, 2, 1)
    we1 = We1.transpose(2, 0, 1)
    we2 = We2.transpose(2, 0, 1)
    wfit = Wfi.T
    wfot = Wfo.T
    bfip = bfi.reshape(1, F)
    wd1 = Wd1.transpose(2, 1, 0)
    wd2 = Wd2.transpose(2, 1, 0)

    def c2(v):
        return v.reshape(-1, 1)

    full = lambda s: pl.BlockSpec(s, lambda b: (0,) * len(s))
    out_y, out_loss, out_perp = pl.pallas_call(
        _body,
        grid=(B // BB,),
        in_specs=[
            pl.BlockSpec((BB, 4, C, TQ), lambda b: (b, 0, 0, 0)),
            full((4, H, C)), full((H, 1)),
            full((4, H, H)), full((H, 1)),
            full((H, F)), full((1, F)),
            full((F, H)), full((1, H)),
            full((4, NE, H)), full((4, H, NE)),
            full((4, H, H)), full((H, 1)),
            full((4, C, H)), full((C, 1)),
        ],
        out_specs=[
            pl.BlockSpec((BB, 4, C, TQ), lambda b: (b, 0, 0, 0)),
            pl.BlockSpec((1, 1), lambda b: (0, 0)),
            pl.BlockSpec((1, 1), lambda b: (0, 0)),
        ],
        out_shape=[
            jax.ShapeDtypeStruct((B, 4, C, TQ), jnp.float32),
            jax.ShapeDtypeStruct((1, 1), jnp.float32),
            jax.ShapeDtypeStruct((1, 1), jnp.float32),
        ],
        scratch_shapes=[
            pltpu.VMEM((4, 1, NE), jnp.float32),
            pltpu.VMEM((1, 1), jnp.float32),
        ],
    )(xp, we1, c2(be1), we2, c2(be2), wfit, bfip,
      wfot, bfo.reshape(1, H), e_all, et_all, wd1, c2(bd1), wd2, c2(bd2))

    y = out_y.transpose(0, 2, 3, 1).reshape(B, C, T)
    return (y, out_loss[0, 0], out_perp[0, 0])
